# baseline (speedup 1.0000x reference)
import functools
import jax
import jax.numpy as jnp
from jax import lax
from jax.experimental import pallas as pl
from jax.experimental.pallas import tpu as pltpu
from jax.experimental.pallas import tpu_sc as plsc


def kernel(x, pe_table):
    B, S, D = x.shape
    NC, NS = 2, 16
    NW = NC * NS
    rows_w = S // NW
    C = 16
    n_chunks = rows_w // C
    nvec = D // 16

    mesh = plsc.VectorSubcoreMesh(core_axis_name="c", subcore_axis_name="s",
                                  num_cores=NC, num_subcores=NS)

    @functools.partial(
        pl.kernel,
        out_type=jax.ShapeDtypeStruct((B, S, D), jnp.float32),
        mesh=mesh,
        scratch_types=[
            pltpu.VMEM((C, D), jnp.float32),
            pltpu.VMEM((B, C, D), jnp.float32),
        ],
    )
    def sc_add(x_hbm, pe_hbm, out_hbm, pe_v, x_v):
        wid = lax.axis_index("s") * NC + lax.axis_index("c")
        base = wid * rows_w

        def chunk_body(ci, carry):
            row0 = base + ci * C
            pltpu.sync_copy(pe_hbm.at[pl.ds(row0, C)], pe_v)
            for b in range(B):
                pltpu.sync_copy(x_hbm.at[b, pl.ds(row0, C)], x_v.at[b])

            def row_body(r, carry2):
                def vec_body(j, carry3):
                    col = j * 16
                    pe_vec = pe_v[r, pl.ds(col, 16)]
                    for b in range(B):
                        x_v[b, r, pl.ds(col, 16)] = x_v[b, r, pl.ds(col, 16)] + pe_vec
                    return carry3
                return lax.fori_loop(0, nvec, vec_body, carry2)
            lax.fori_loop(0, C, row_body, 0)

            for b in range(B):
                pltpu.sync_copy(x_v.at[b], out_hbm.at[b, pl.ds(row0, C)])
            return carry
        lax.fori_loop(0, n_chunks, chunk_body, 0)

    return sc_add(x, pe_table)


# speedup vs baseline: 1.2303x; 1.2303x over previous
import functools
import jax
import jax.numpy as jnp
from jax import lax
from jax.experimental import pallas as pl
from jax.experimental.pallas import tpu as pltpu
from jax.experimental.pallas import tpu_sc as plsc


def kernel(x, pe_table):
    B, S, D = x.shape
    NC, NS = 2, 16
    NW = NC * NS
    rows_w = S // NW
    C = 8
    n_chunks = rows_w // C
    nvec = D // 16

    mesh = plsc.VectorSubcoreMesh(core_axis_name="c", subcore_axis_name="s",
                                  num_cores=NC, num_subcores=NS)

    @functools.partial(
        pl.kernel,
        out_type=jax.ShapeDtypeStruct((B, S, D), jnp.float32),
        mesh=mesh,
        scratch_types=[
            pltpu.VMEM((2, C, D), jnp.float32),
            pltpu.VMEM((2, B, C, D), jnp.float32),
            pltpu.SemaphoreType.DMA((2,)),
            pltpu.SemaphoreType.DMA((2,)),
        ],
    )
    def sc_add(x_hbm, pe_hbm, out_hbm, pe_v, x_v, in_sems, out_sems):
        wid = lax.axis_index("s") * NC + lax.axis_index("c")
        base = wid * rows_w

        def start_in(ci, t):
            row0 = base + ci * C
            pltpu.async_copy(pe_hbm.at[pl.ds(row0, C)], pe_v.at[t],
                             in_sems.at[t])
            for b in range(B):
                pltpu.async_copy(x_hbm.at[b, pl.ds(row0, C)], x_v.at[t, b],
                                 in_sems.at[t])

        def wait_in(s):
            pltpu.make_async_copy(pe_hbm.at[pl.ds(0, C)], pe_v.at[s],
                                  in_sems.at[s]).wait()
            for b in range(B):
                pltpu.make_async_copy(x_hbm.at[b, pl.ds(0, C)], x_v.at[s, b],
                                      in_sems.at[s]).wait()

        def start_out(ci, s):
            row0 = base + ci * C
            for b in range(B):
                pltpu.async_copy(x_v.at[s, b], out_hbm.at[b, pl.ds(row0, C)],
                                 out_sems.at[s])

        def wait_out(s):
            for b in range(B):
                pltpu.make_async_copy(x_v.at[s, b], out_hbm.at[b, pl.ds(0, C)],
                                      out_sems.at[s]).wait()

        start_in(0, 0)

        def chunk_body(ci, carry):
            s = ci & 1
            t = 1 - s

            @pl.when(ci + 1 < n_chunks)
            def _():
                @pl.when(ci >= 1)
                def _():
                    wait_out(t)
                start_in(ci + 1, t)

            wait_in(s)

            def vec_body(i, c3):
                r = i >> 6
                col = (i & (nvec - 1)) * 16
                pe_vec = pe_v[s, r, pl.ds(col, 16)]
                for b in range(B):
                    x_v[s, b, r, pl.ds(col, 16)] = (
                        x_v[s, b, r, pl.ds(col, 16)] + pe_vec)
                return c3
            lax.fori_loop(0, C * nvec, vec_body, 0)

            start_out(ci, s)
            return carry

        lax.fori_loop(0, n_chunks, chunk_body, 0)
        wait_out(0)
        wait_out(1)

    return sc_add(x, pe_table)
